# Initial kernel scaffold; baseline (speedup 1.0000x reference)
import jax
import jax.numpy as jnp
from jax.experimental import pallas as pl


def kernel(x, type_vec):
    raise NotImplementedError("write your pallas kernel here")


# speedup vs baseline: 14.3300x; 14.3300x over previous
import functools

import jax
import jax.numpy as jnp
from jax.experimental import pallas as pl
from jax.experimental.pallas import tpu as pltpu

N = 320000
D = 128
NUM_TYPES = 8
EPS = 1e-05

BLK = 3200
NB = N // BLK


def _stats_body(type_ref, x_ref, mean_ref, rstd_ref, s1_ref, s2_ref, cnt_ref):
    j = pl.program_id(0)

    @pl.when(j == 0)
    def _init():
        s1_ref[...] = jnp.zeros_like(s1_ref)
        s2_ref[...] = jnp.zeros_like(s2_ref)
        cnt_ref[...] = jnp.zeros_like(cnt_ref)

    t = type_ref[0, 0, :]
    xb = x_ref[...]
    oh = (jax.lax.broadcasted_iota(jnp.int32, (NUM_TYPES, BLK), 0)
          == t[None, :]).astype(jnp.float32)
    dn = (((1,), (0,)), ((), ()))
    s1_ref[...] += jax.lax.dot_general(oh, xb, dn,
                                       preferred_element_type=jnp.float32)
    s2_ref[...] += jax.lax.dot_general(oh, xb * xb, dn,
                                       preferred_element_type=jnp.float32)
    cnt_ref[0, :] += jnp.sum(oh, axis=1)

    @pl.when(j == NB - 1)
    def _finalize():
        c = jnp.maximum(cnt_ref[0, :], 1.0)[:, None]
        mean = s1_ref[...] / c
        var = s2_ref[...] / c - mean * mean
        var = jnp.maximum(var, 0.0)
        mean_ref[...] = mean
        rstd_ref[...] = jax.lax.rsqrt(var + EPS)


def _norm_body(type_ref, x_ref, mean_ref, rstd_ref, out_ref):
    t = type_ref[0, 0, :]
    oh = (t[:, None] == jax.lax.broadcasted_iota(
        jnp.int32, (BLK, NUM_TYPES), 1)).astype(jnp.float32)
    dn = (((1,), (0,)), ((), ()))
    m = jax.lax.dot_general(oh, mean_ref[...], dn,
                            preferred_element_type=jnp.float32)
    r = jax.lax.dot_general(oh, rstd_ref[...], dn,
                            preferred_element_type=jnp.float32)
    out_ref[...] = (x_ref[...] - m) * r


@jax.jit
def kernel(x, type_vec):
    tv3 = type_vec.astype(jnp.int32).reshape(NB, 1, BLK)

    mean, rstd = pl.pallas_call(
        _stats_body,
        grid=(NB,),
        in_specs=[
            pl.BlockSpec((1, 1, BLK), lambda j: (j, 0, 0)),
            pl.BlockSpec((BLK, D), lambda j: (j, 0)),
        ],
        out_specs=[
            pl.BlockSpec((NUM_TYPES, D), lambda j: (0, 0)),
            pl.BlockSpec((NUM_TYPES, D), lambda j: (0, 0)),
        ],
        out_shape=[
            jax.ShapeDtypeStruct((NUM_TYPES, D), jnp.float32),
            jax.ShapeDtypeStruct((NUM_TYPES, D), jnp.float32),
        ],
        scratch_shapes=[
            pltpu.VMEM((NUM_TYPES, D), jnp.float32),
            pltpu.VMEM((NUM_TYPES, D), jnp.float32),
            pltpu.VMEM((1, NUM_TYPES), jnp.float32),
        ],
    )(tv3, x)

    out = pl.pallas_call(
        _norm_body,
        grid=(NB,),
        in_specs=[
            pl.BlockSpec((1, 1, BLK), lambda j: (j, 0, 0)),
            pl.BlockSpec((BLK, D), lambda j: (j, 0)),
            pl.BlockSpec((NUM_TYPES, D), lambda j: (0, 0)),
            pl.BlockSpec((NUM_TYPES, D), lambda j: (0, 0)),
        ],
        out_specs=pl.BlockSpec((BLK, D), lambda j: (j, 0)),
        out_shape=jax.ShapeDtypeStruct((N, D), jnp.float32),
    )(tv3, x, mean, rstd)
    return out
